# baseline (speedup 1.0000x reference)
import functools

import jax
import jax.numpy as jnp
from jax import lax
from jax.experimental import pallas as pl
from jax.experimental.pallas import tpu as pltpu
from jax.experimental.pallas import tpu_sc as plsc

_B, _T, _D = 8, 4096, 1024


def _take_last_scs(x_hbm, seq_hbm, out_hbm, seq_s):
    cid = lax.axis_index("c")

    @pl.when(cid == 0)
    def _():
        pltpu.sync_copy(seq_hbm, seq_s)
        for b in range(_B):
            row = (seq_s[b] - 1) & jnp.int32(_T - 1)
            pltpu.sync_copy(
                x_hbm.at[b, pl.ds(row, 1)], out_hbm.at[pl.ds(b, 1)]
            )


@jax.jit
def kernel(x, seq_len):
    seq = seq_len.astype(jnp.int32)
    mesh = plsc.ScalarSubcoreMesh(axis_name="c", num_cores=1)
    run = functools.partial(
        pl.kernel,
        mesh=mesh,
        out_type=jax.ShapeDtypeStruct((_B, _D), jnp.float32),
        scratch_types=[pltpu.SMEM((_B,), jnp.int32)],
    )(_take_last_scs)
    return run(x, seq)


# speedup vs baseline: 1.3849x; 1.3849x over previous
import functools

import jax
import jax.numpy as jnp
from jax import lax
from jax.experimental import pallas as pl
from jax.experimental.pallas import tpu as pltpu
from jax.experimental.pallas import tpu_sc as plsc

_B, _T, _D = 8, 4096, 1024


def _take_last_scs(x_hbm, seq_hbm, out_hbm, seq_s, sem):
    cid = lax.axis_index("c")

    @pl.when(cid == 0)
    def _():
        pltpu.sync_copy(seq_hbm, seq_s)
        copies = []
        for b in range(_B):
            row = (seq_s[b] - 1) & jnp.int32(_T - 1)
            copies.append(
                pltpu.async_copy(
                    x_hbm.at[b, pl.ds(row, 1)], out_hbm.at[pl.ds(b, 1)], sem
                )
            )
        for c in copies:
            c.wait()


@jax.jit
def kernel(x, seq_len):
    seq = seq_len.astype(jnp.int32)
    mesh = plsc.ScalarSubcoreMesh(axis_name="c", num_cores=1)
    run = functools.partial(
        pl.kernel,
        mesh=mesh,
        out_type=jax.ShapeDtypeStruct((_B, _D), jnp.float32),
        scratch_types=[pltpu.SMEM((_B,), jnp.int32), pltpu.SemaphoreType.DMA],
    )(_take_last_scs)
    return run(x, seq)


# speedup vs baseline: 1.3951x; 1.0074x over previous
import functools

import jax
import jax.numpy as jnp
from jax import lax
from jax.experimental import pallas as pl
from jax.experimental.pallas import tpu as pltpu
from jax.experimental.pallas import tpu_sc as plsc

_B, _T, _D = 8, 4096, 1024


def _take_last_scs(x_hbm, seq_hbm, out_hbm, seq_s, sem):
    cid = lax.axis_index("c")

    @pl.when(cid == 0)
    def _():
        pltpu.sync_copy(seq_hbm, seq_s)
        copies = []
        for b in range(_B):
            row = (seq_s[b] - 1) & jnp.int32(_T - 1)
            copies.append(
                pltpu.async_copy(
                    x_hbm.at[b, pl.ds(row, 1)], out_hbm.at[pl.ds(b, 1)], sem
                )
            )
        for c in copies:
            c.wait()


@jax.jit
def kernel(x, seq_len):
    seq = seq_len.astype(jnp.int32)
    mesh = plsc.ScalarSubcoreMesh(axis_name="c", num_cores=1)
    run = functools.partial(
        pl.kernel,
        mesh=mesh,
        out_type=jax.ShapeDtypeStruct((_B, _D), jnp.float32),
        scratch_types=[pltpu.SMEM((_B,), jnp.int32), pltpu.SemaphoreType.DMA],
        compiler_params=pltpu.CompilerParams(skip_device_barrier=True),
    )(_take_last_scs)
    return run(x, seq)
